# baseline (speedup 1.0000x reference)
import functools

import jax
import jax.numpy as jnp
from jax import lax
from jax.experimental import pallas as pl
from jax.experimental.pallas import tpu as pltpu
from jax.experimental.pallas import tpu_sc as plsc

_K = 1024
_D = 64
_BLK = 2048
_COMMIT = 0.25


def _vq_body(x_ref, w_ref, enc_ref, idx_ref, loss_ref, ppl_ref,
             sse_acc, cnt_acc, *, n_tokens, n_blocks):
    i = pl.program_id(0)
    x = x_ref[...]
    w = w_ref[...]
    m2 = jax.lax.dot_general(x * (-2.0), w, (((1,), (1,)), ((), ())),
                             preferred_element_type=jnp.float32)
    xsq = jnp.sum(x * x, axis=1, keepdims=True)
    esq = jnp.sum(w * w, axis=1)[None, :]
    d = (xsq + esq) + m2
    dmin = jnp.min(d, axis=1, keepdims=True)
    col = jax.lax.broadcasted_iota(jnp.int32, d.shape, 1)
    idx = jnp.min(jnp.where(d == dmin, col, _K), axis=1, keepdims=True)
    idx_ref[...] = idx
    mask = (d == dmin).astype(jnp.float32)
    cnt = jnp.sum(mask, axis=0, keepdims=True)
    total = jnp.sum(cnt)
    enc_ref[...] = mask

    @pl.when(i == 0)
    def _init():
        sse_acc[0, 0] = jnp.sum(dmin)
        cnt_acc[...] = cnt

    @pl.when(i > 0)
    def _accum():
        sse_acc[0, 0] += jnp.sum(dmin)
        cnt_acc[...] += cnt

    @pl.when(total != _BLK)
    def _fix_ties():
        one_hot = (col == idx).astype(jnp.float32)
        enc_ref[...] = one_hot
        fixed = jnp.sum(one_hot, axis=0, keepdims=True)
        cnt_acc[...] += fixed - cnt

    @pl.when(i == n_blocks - 1)
    def _finalize():
        mse = sse_acc[0, 0] / (n_tokens * _D)
        loss_ref[...] = jnp.full((1, 1), (1.0 + _COMMIT) * mse, jnp.float32)
        avg = cnt_acc[...] * (1.0 / n_tokens)
        ent = -jnp.sum(avg * jnp.log(avg + 1e-10), keepdims=True)
        ppl_ref[...] = jnp.exp(ent)


def _tc_pass(flat, W, n, n_blocks):
    body = functools.partial(_vq_body, n_tokens=n, n_blocks=n_blocks)
    return pl.pallas_call(
        body,
        grid=(n_blocks,),
        in_specs=[
            pl.BlockSpec((_BLK, _D), lambda i: (i, 0)),
            pl.BlockSpec((_K, _D), lambda i: (0, 0)),
        ],
        out_specs=[
            pl.BlockSpec((_BLK, _K), lambda i: (i, 0)),
            pl.BlockSpec((_BLK, 1), lambda i: (i, 0)),
            pl.BlockSpec((1, 1), lambda i: (0, 0)),
            pl.BlockSpec((1, 1), lambda i: (0, 0)),
        ],
        out_shape=[
            jax.ShapeDtypeStruct((n, _K), jnp.float32),
            jax.ShapeDtypeStruct((n, 1), jnp.int32),
            jax.ShapeDtypeStruct((1, 1), jnp.float32),
            jax.ShapeDtypeStruct((1, 1), jnp.float32),
        ],
        scratch_shapes=[
            pltpu.SMEM((1, 1), jnp.float32),
            pltpu.VMEM((1, _K), jnp.float32),
        ],
    )(flat, W)


def _sc_gather(W128, idx, n):
    info = plsc.get_sparse_core_info()
    nw = info.num_cores * info.num_subcores
    b_per_w = n // nw
    chunk = 512
    n_chunks = b_per_w // chunk
    mesh = plsc.VectorSubcoreMesh(core_axis_name="c", subcore_axis_name="s")

    @functools.partial(
        pl.kernel, mesh=mesh,
        out_type=jax.ShapeDtypeStruct((n, 2 * _D), jnp.float32),
        scratch_types=[
            pltpu.VMEM((chunk,), jnp.int32),
            pltpu.VMEM((chunk, 2 * _D), jnp.float32),
            pltpu.SemaphoreType.DMA,
        ],
    )
    def gather_kernel(table_hbm, idx_hbm, out_hbm, idx_v, rows_v, sem):
        wid = lax.axis_index("s") * info.num_cores + lax.axis_index("c")
        for c in range(n_chunks):
            base = wid * b_per_w + c * chunk
            pltpu.sync_copy(idx_hbm.at[pl.ds(base, chunk)], idx_v)
            pltpu.async_copy(table_hbm.at[idx_v], rows_v, sem).wait()
            pltpu.sync_copy(rows_v, out_hbm.at[pl.ds(base, chunk)])

    return gather_kernel(W128, idx)


def kernel(inputs, W):
    input_shape = inputs.shape
    flat = inputs.reshape(-1, _D)
    n = flat.shape[0]
    n_blocks = n // _BLK
    enc, idx, loss, ppl = _tc_pass(flat, W, n, n_blocks)
    W128 = jnp.pad(W, ((0, 0), (0, 2 * _D - W.shape[1])))
    q = _sc_gather(W128, idx.reshape(n), n)[:, :_D]
    return (loss[0, 0], q.reshape(input_shape), ppl[0, 0], enc)
